# baseline (speedup 1.0000x reference)
import functools

import jax
import jax.numpy as jnp
from jax import lax
from jax.experimental import pallas as pl
from jax.experimental.pallas import tpu as pltpu
from jax.experimental.pallas import tpu_sc as plsc

N_REGIONS = 64
TEXT_LEN = 512
VOCAB = 256
D_MODEL = 128

NC = 2
NS = 16
NW = NC * NS
RPW = N_REGIONS // NW
LANES = 16
SUB = VOCAB * LANES


def _sc_hist_body(ids_hbm, counts_hbm, ids_v, c16_v, out_v):
    wid = lax.axis_index("s") * NC + lax.axis_index("c")
    lane = lax.broadcasted_iota(jnp.int32, (LANES,), 0)
    zeros = jnp.zeros((LANES,), jnp.float32)
    ones = jnp.ones((LANES,), jnp.float32)

    pltpu.sync_copy(ids_hbm.at[pl.ds(wid * (RPW * TEXT_LEN), RPW * TEXT_LEN)],
                    ids_v)

    def zero_step(i, c):
        for u in range(8):
            c16_v[pl.ds((i * 8 + u) * LANES, LANES)] = zeros
        return c
    lax.fori_loop(0, RPW * VOCAB // 8, zero_step, 0)

    for j in range(RPW):
        def scat_step(g, c, j=j):
            for u in range(4):
                idv = ids_v[pl.ds(j * TEXT_LEN + (g * 4 + u) * LANES, LANES)]
                plsc.addupdate_scatter(
                    c16_v, [idv * LANES + lane + (j * SUB)], ones)
            return c
        lax.fori_loop(0, TEXT_LEN // LANES // 4, scat_step, 0)

    lane16 = lane * LANES
    for j in range(RPW):
        def red_step(t, c, j=j):
            base = j * SUB + t * (LANES * LANES)
            acc = zeros
            for off in range(LANES):
                acc = acc + plsc.load_gather(c16_v, [lane16 + (base + off)])
            out_v[pl.ds(j * VOCAB + t * LANES, LANES)] = acc
            return c
        lax.fori_loop(0, VOCAB // LANES, red_step, 0)

    pltpu.sync_copy(out_v, counts_hbm.at[pl.ds(wid * (RPW * VOCAB),
                                               RPW * VOCAB)])


def _sc_histogram(ids_flat):
    mesh = plsc.VectorSubcoreMesh(core_axis_name="c", subcore_axis_name="s",
                                  num_cores=NC, num_subcores=NS)
    f = functools.partial(
        pl.kernel,
        out_type=jax.ShapeDtypeStruct((N_REGIONS * VOCAB,), jnp.float32),
        mesh=mesh,
        scratch_types=[
            pltpu.VMEM((RPW * TEXT_LEN,), jnp.int32),
            pltpu.VMEM((RPW * SUB,), jnp.float32),
            pltpu.VMEM((RPW * VOCAB,), jnp.float32),
        ],
        compiler_params=pltpu.CompilerParams(needs_layout_passes=False),
    )(_sc_hist_body)
    return f(ids_flat)


def _tc_matmul_body(counts_ref, w_ref, out_ref):
    out_ref[...] = jnp.dot(counts_ref[...], w_ref[...],
                           preferred_element_type=jnp.float32) * (1.0 / TEXT_LEN)


def kernel(ids, embed_weight):
    counts = _sc_histogram(ids.reshape(-1)).reshape(N_REGIONS, VOCAB)
    return pl.pallas_call(
        _tc_matmul_body,
        out_shape=jax.ShapeDtypeStruct((N_REGIONS, D_MODEL), jnp.float32),
        in_specs=[
            pl.BlockSpec(memory_space=pltpu.VMEM),
            pl.BlockSpec(memory_space=pltpu.VMEM),
        ],
        out_specs=pl.BlockSpec(memory_space=pltpu.VMEM),
    )(counts, embed_weight)
